# Initial kernel scaffold; baseline (speedup 1.0000x reference)
import jax
import jax.numpy as jnp
from jax.experimental import pallas as pl


def kernel(x, w1, b1, g1, be1, w2, b2, g2, be2):
    raise NotImplementedError("write your pallas kernel here")


# speedup vs baseline: 1.1430x; 1.1430x over previous
import jax
import jax.numpy as jnp
from jax.experimental import pallas as pl

EPS = 1e-5
DET_THRESH = 0.015
B = 8
HW = 4096
CIN = 128
CMID = 256
CPAD = 128


def _conv1_body(x_ref, w_ref, h_ref, st_ref):
    acc = jnp.zeros((HW, CMID), jnp.float32)
    for dh in range(3):
        for dw in range(3):
            xs = x_ref[0, dh:dh + 64, dw:dw + 64, :].reshape(HW, CIN)
            acc = acc + jnp.dot(xs, w_ref[dh * 3 + dw],
                                preferred_element_type=jnp.float32)
    h_ref[0] = acc
    st_ref[0, 0] = jnp.sum(acc, axis=0)
    st_ref[0, 1] = jnp.sum(acc * acc, axis=0)


def _conv2_body(h_ref, bn1_ref, w2_ref, l_ref, st_ref):
    h = h_ref[0]
    r = jnp.maximum(h * bn1_ref[0][None, :] + bn1_ref[1][None, :], 0.0)
    l = jnp.dot(r.astype(jnp.bfloat16), w2_ref[...],
                preferred_element_type=jnp.float32)
    l_ref[0] = l
    st_ref[0, 0] = jnp.sum(l, axis=0)
    st_ref[0, 1] = jnp.sum(l * l, axis=0)


def _final_body(l_ref, bn2_ref, logits_ref, prob_ref, pred_ref):
    l = l_ref[0] * bn2_ref[0][None, :] + bn2_ref[1][None, :]
    lt = l.T
    logits_ref[0] = lt[:65]
    p = l[:, :64].reshape(64, 64, 8, 8).transpose(0, 2, 1, 3).reshape(512, 512)
    prob_ref[0] = p
    pred_ref[0] = (p >= DET_THRESH).astype(jnp.int32)


def kernel(x, w1, b1, g1, be1, w2, b2, g2, be2):
    xh = x.transpose(0, 2, 3, 1).astype(jnp.bfloat16)
    xp = jnp.pad(xh, ((0, 0), (1, 1), (1, 1), (0, 0)))
    w9 = w1.transpose(2, 3, 1, 0).reshape(9, CIN, CMID).astype(jnp.bfloat16)

    h, st1 = pl.pallas_call(
        _conv1_body,
        grid=(B,),
        in_specs=[
            pl.BlockSpec((1, 66, 66, CIN), lambda b: (b, 0, 0, 0)),
            pl.BlockSpec((9, CIN, CMID), lambda b: (0, 0, 0)),
        ],
        out_specs=[
            pl.BlockSpec((1, HW, CMID), lambda b: (b, 0, 0)),
            pl.BlockSpec((1, 2, CMID), lambda b: (b, 0, 0)),
        ],
        out_shape=[
            jax.ShapeDtypeStruct((B, HW, CMID), jnp.float32),
            jax.ShapeDtypeStruct((B, 2, CMID), jnp.float32),
        ],
    )(xp, w9)

    n = B * HW
    mean1 = jnp.sum(st1[:, 0], axis=0) / n
    var1 = jnp.sum(st1[:, 1], axis=0) / n - mean1 * mean1
    scale1 = g1 * jax.lax.rsqrt(var1 + EPS)
    shift1 = be1 - mean1 * scale1
    bn1 = jnp.stack([scale1, shift1])

    w2p = jnp.zeros((CMID, CPAD), jnp.float32)
    w2p = w2p.at[:, :65].set(w2[:, :, 0, 0].T).astype(jnp.bfloat16)

    l_raw, st2 = pl.pallas_call(
        _conv2_body,
        grid=(B,),
        in_specs=[
            pl.BlockSpec((1, HW, CMID), lambda b: (b, 0, 0)),
            pl.BlockSpec((2, CMID), lambda b: (0, 0)),
            pl.BlockSpec((CMID, CPAD), lambda b: (0, 0)),
        ],
        out_specs=[
            pl.BlockSpec((1, HW, CPAD), lambda b: (b, 0, 0)),
            pl.BlockSpec((1, 2, CPAD), lambda b: (b, 0, 0)),
        ],
        out_shape=[
            jax.ShapeDtypeStruct((B, HW, CPAD), jnp.float32),
            jax.ShapeDtypeStruct((B, 2, CPAD), jnp.float32),
        ],
    )(h, bn1, w2p)

    mean2 = jnp.sum(st2[:, 0], axis=0) / n
    var2 = jnp.sum(st2[:, 1], axis=0) / n - mean2 * mean2
    mask = jnp.arange(CPAD) < 65
    g2p = jnp.pad(g2, (0, CPAD - 65))
    be2p = jnp.pad(be2, (0, CPAD - 65))
    scale2 = jnp.where(mask, g2p * jax.lax.rsqrt(var2 + EPS), 0.0)
    shift2 = jnp.where(mask, be2p - mean2 * scale2, 0.0)
    bn2 = jnp.stack([scale2, shift2])

    logits, prob, pred = pl.pallas_call(
        _final_body,
        grid=(B,),
        in_specs=[
            pl.BlockSpec((1, HW, CPAD), lambda b: (b, 0, 0)),
            pl.BlockSpec((2, CPAD), lambda b: (0, 0)),
        ],
        out_specs=[
            pl.BlockSpec((1, 65, HW), lambda b: (b, 0, 0)),
            pl.BlockSpec((1, 512, 512), lambda b: (b, 0, 0)),
            pl.BlockSpec((1, 512, 512), lambda b: (b, 0, 0)),
        ],
        out_shape=[
            jax.ShapeDtypeStruct((B, 65, HW), jnp.float32),
            jax.ShapeDtypeStruct((B, 512, 512), jnp.float32),
            jax.ShapeDtypeStruct((B, 512, 512), jnp.int32),
        ],
    )(l_raw, bn2)

    return (logits.reshape(B, 65, 64, 64), prob, pred)
